# baseline (speedup 1.0000x reference)
import jax
import jax.numpy as jnp
from jax import lax
from jax.experimental import pallas as pl
from jax.experimental.pallas import tpu as pltpu
from jax.experimental.pallas import tpu_sc as plsc

_N = 10000
_E = 160000

_NC = 2
_NS = 16
_CHUNK = 128
_EPAD = 163840
_NPAD = 10112
_RPT = _NPAD // _NS
_DUMMY = _NPAD - 8
_NCHS = 20

_BM = 1000
_BMF = 2000


def _make_sc_aggregate(wc: int, n_stages: int, core_split: bool,
                       with_deg: bool, chunk: int, nchs: int, depth: int,
                       async_scat: bool = False):
    mesh = plsc.VectorSubcoreMesh(
        core_axis_name="c", subcore_axis_name="s",
        num_cores=_NC, num_subcores=_NS)
    agg_t = jax.ShapeDtypeStruct((_NC, _NPAD, wc), jnp.float32)
    out_type = ([agg_t, jax.ShapeDtypeStruct((_NC, _NPAD, 16), jnp.float32)]
                if with_deg else agg_t)
    scratch = [
        pltpu.VMEM_SHARED((_NPAD, wc), jnp.float32),
        pltpu.VMEM((nchs, chunk), jnp.int32),
        pltpu.VMEM((nchs, chunk), jnp.int32),
    ]
    scratch += [pltpu.VMEM((chunk, wc), jnp.float32) for _ in range(depth)]
    scratch += [pltpu.SemaphoreType.DMA for _ in range(depth)]
    if async_scat:
        scratch.append(pltpu.SemaphoreType.DMA)
    if with_deg:
        scratch += [
            pltpu.VMEM_SHARED((_NPAD, 16), jnp.float32),
            pltpu.VMEM((chunk, 16), jnp.float32),
        ]

    def body(*args):
        if with_deg:
            (z_agg, z_deg, ones, table, src, dst, agg_out, deg_out,
             agg_sh, src_v, dst_v, *rest) = args
            bufs, sems = rest[:depth], rest[depth:2 * depth]
            deg_sh, ones_v = rest[2 * depth:]
        else:
            (z_agg, table, src, dst, agg_out,
             agg_sh, src_v, dst_v, *rest) = args
            bufs, sems = rest[:depth], rest[depth:2 * depth]
            sem_s = rest[2 * depth] if async_scat else None
        cid = lax.axis_index("c")
        sid = lax.axis_index("s")
        row0 = sid * _RPT

        pltpu.sync_copy(z_agg, agg_sh.at[pl.ds(row0, _RPT), :])
        if with_deg:
            pltpu.sync_copy(z_deg, deg_sh.at[pl.ds(row0, _RPT), :])
            pltpu.sync_copy(ones, ones_v)
        plsc.subcore_barrier()

        def fire(j, k):
            pltpu.async_copy(table.at[src_v.at[j]], bufs[k], sems[k])

        def drain(k):
            pltpu.make_async_copy(
                table.at[src_v.at[0]], bufs[k], sems[k]).wait()

        def scat(j, k, deg_on):
            pltpu.sync_copy(bufs[k], agg_sh.at[dst_v.at[j]], add=True)
            if with_deg:
                @pl.when(deg_on)
                def _():
                    pltpu.sync_copy(ones_v, deg_sh.at[dst_v.at[j]], add=True)

        for q in range(n_stages):
            pltpu.sync_copy(src.at[cid, sid, q], src_v)
            if core_split:
                pltpu.sync_copy(dst.at[cid, sid, q], dst_v)
            else:
                pltpu.sync_copy(dst.at[sid, q], dst_v)
            deg_on = q // max(1, n_stages // _NC) == cid

            if async_scat:
                def fire_s(j, k):
                    pltpu.async_copy(bufs[k], agg_sh.at[dst_v.at[j]],
                                     sem_s, add=True)

                def drain_s():
                    pltpu.make_async_copy(
                        bufs[0], agg_sh.at[dst_v.at[0]], sem_s).wait()

                fire(0, 0)
                fire(1, 1)
                for i in range(2):
                    fire(i + 2, i + 2)
                    drain(i)
                    fire_s(i, i)

                @pl.loop(0, (nchs - 4) // 4)
                def _(p):
                    for k in range(4):
                        ki = (2 + k) % 4
                        drain_s()
                        fire(4 * p + 4 + k, k)
                        drain(ki)
                        fire_s(4 * p + 2 + k, ki)

                for k in range(2):
                    ki = (2 + k) % 4
                    drain_s()
                    drain(ki)
                    fire_s(nchs - 2 + k, ki)
                drain_s()
                drain_s()
            else:
                for k in range(depth - 1):
                    fire(k, k)

                @pl.loop(0, nchs // depth - 1)
                def _(p):
                    for k in range(depth):
                        j = depth * p + k
                        fire(j + depth - 1, (k + depth - 1) % depth)
                        drain(k)
                        scat(j, k, deg_on)

                fire(nchs - 1, depth - 1)
                for k in range(depth):
                    drain(k)
                    scat(nchs - depth + k, k, deg_on)

        plsc.subcore_barrier()
        pltpu.sync_copy(agg_sh.at[pl.ds(row0, _RPT), :],
                        agg_out.at[cid, pl.ds(row0, _RPT), :])
        if with_deg:
            pltpu.sync_copy(deg_sh.at[pl.ds(row0, _RPT), :],
                            deg_out.at[cid, pl.ds(row0, _RPT), :])

    return pl.kernel(
        body, out_type=out_type, mesh=mesh, scratch_types=scratch,
        name=f"sc_agg_w{wc}",
        compiler_params=pltpu.CompilerParams(use_tc_tiling_on_sc=False))


_sc_agg_l1 = _make_sc_aggregate(144, 8, False, False, 64, 20, 4,
                                async_scat=True)
_sc_agg_l2 = _make_sc_aggregate(32, 4, False, False, _CHUNK, _NCHS, 4,
                                async_scat=True)


def _rowspec(w, bm=_BM):
    return pl.BlockSpec((bm, w), lambda i: (i, 0))


def _pairspec(w, bm=_BM):
    return pl.BlockSpec((_NC, bm, w), lambda i: (0, i, 0))


def _full(shape):
    return pl.BlockSpec(shape, lambda i: (0, 0))


def _tcs_body(x_ref, ws_ref, b_ref, o_ref):
    o_ref[...] = jnp.dot(x_ref[...], ws_ref[...],
                         preferred_element_type=jnp.float32) + b_ref[...]


_tc_self1 = pl.pallas_call(
    _tcs_body,
    grid=(_N // _BM,),
    in_specs=[_rowspec(256), _full((256, 256)), _full((1, 256))],
    out_specs=_rowspec(256),
    out_shape=jax.ShapeDtypeStruct((_N, 256), jnp.float32),
)


def _tca_body(xs_ref, agg_ref, wn0_ref, wn1_ref, ws2_ref, wn2_ref,
              b2_ref, hs_ref, hw_ref):
    r = 1.0 / jnp.maximum(agg_ref[0, :, 128:129], 1.0)
    acc = xs_ref[...]
    acc += jnp.dot(agg_ref[0, :, :128] * r, wn0_ref[...],
                   preferred_element_type=jnp.float32)
    acc += jnp.dot(agg_ref[1, :, :128] * r, wn1_ref[...],
                   preferred_element_type=jnp.float32)
    h = jnp.maximum(acc, 0.0)
    hs_ref[...] = jnp.dot(h, ws2_ref[...],
                          preferred_element_type=jnp.float32) + b2_ref[...]
    hw_ref[...] = jnp.dot(h, wn2_ref[...], preferred_element_type=jnp.float32)


_tc_mid = pl.pallas_call(
    _tca_body,
    grid=(_N // _BM,),
    in_specs=[_rowspec(256), _pairspec(144),
              _full((128, 256)), _full((128, 256)),
              _full((256, 64)), _full((256, 64)), _full((1, 64))],
    out_specs=[_rowspec(64), _rowspec(64)],
    out_shape=[jax.ShapeDtypeStruct((_N, 64), jnp.float32),
               jax.ShapeDtypeStruct((_N, 64), jnp.float32)],
)


def _tcf_body(hs_ref, agg_ref, deg_ref, o_ref):
    r = 1.0 / jnp.maximum(deg_ref[0, :, 128:129], 1.0)
    o_ref[...] = hs_ref[...] + jnp.concatenate(
        [agg_ref[0] * r, agg_ref[1] * r], axis=1)


_tc_final = pl.pallas_call(
    _tcf_body,
    grid=(_N // _BMF,),
    in_specs=[_rowspec(64, _BMF), _pairspec(32, _BMF),
              pl.BlockSpec((1, _BMF, 144), lambda i: (0, i, 0))],
    out_specs=_rowspec(64, _BMF),
    out_shape=jax.ShapeDtypeStruct((_N, 64), jnp.float32),
)


def kernel(x, edge_index, W_self1, W_neigh1, b1, W_self2, W_neigh2, b2):
    src = edge_index[0].astype(jnp.int32)
    dst = edge_index[1].astype(jnp.int32)
    pad = _EPAD - _E
    srcp = jnp.concatenate([src, jnp.zeros((pad,), jnp.int32)])
    pad_dst = _N + jax.lax.rem(jnp.arange(pad, dtype=jnp.int32),
                               jnp.int32(_NPAD - _N))
    dstp = jnp.concatenate([dst, pad_dst])
    base = srcp * 2
    src1 = jnp.stack([base, base + 1]).reshape(_NC, _NS, 8, _NCHS, 64)
    dst1 = dstp.reshape(_NS, 8, _NCHS, 64)
    src3 = jnp.stack([base, base + 1]).reshape(_NC, _NS, 4, _NCHS, _CHUNK)
    dst3 = dstp.reshape(_NS, 4, _NCHS, _CHUNK)

    table1 = jnp.concatenate(
        [x.reshape(2 * _N, 128), jnp.ones((2 * _N, 16), jnp.float32)], axis=1)
    z1 = jnp.zeros((_RPT, 144), jnp.float32)
    agg1 = _sc_agg_l1(z1, table1, src1, dst1)

    xs = _tc_self1(x, W_self1, b1.reshape(1, 256))
    hs, hw = _tc_mid(xs, agg1, W_neigh1[:128], W_neigh1[128:],
                     W_self2, W_neigh2, b2.reshape(1, 64))

    z2 = jnp.zeros((_RPT, 32), jnp.float32)
    table2 = hw.reshape(2 * _N, 32)
    agg2 = _sc_agg_l2(z2, table2, src3, dst3)

    return _tc_final(hs, agg2, agg1)


# speedup vs baseline: 1.1959x; 1.1959x over previous
import jax
import jax.numpy as jnp
from jax import lax
from jax.experimental import pallas as pl
from jax.experimental.pallas import tpu as pltpu
from jax.experimental.pallas import tpu_sc as plsc

_N = 10000
_E = 160000

_NC = 2
_NS = 16
_CHUNK = 128
_EPAD = 163840
_NPAD = 10112
_RPT = _NPAD // _NS
_DUMMY = _NPAD - 8
_NCHS = 20

_BM = 1000
_BMF = 2000


def _make_sc_aggregate(wc: int, n_stages: int, core_split: bool,
                       with_deg: bool, chunk: int, nchs: int, depth: int,
                       async_scat: bool = False):
    mesh = plsc.VectorSubcoreMesh(
        core_axis_name="c", subcore_axis_name="s",
        num_cores=_NC, num_subcores=_NS)
    agg_t = jax.ShapeDtypeStruct((_NC, _NPAD, wc), jnp.float32)
    out_type = ([agg_t, jax.ShapeDtypeStruct((_NC, _NPAD, 16), jnp.float32)]
                if with_deg else agg_t)
    scratch = [
        pltpu.VMEM_SHARED((_NPAD, wc), jnp.float32),
        pltpu.VMEM((nchs, chunk), jnp.int32),
        pltpu.VMEM((nchs, chunk), jnp.int32),
    ]
    scratch += [pltpu.VMEM((chunk, wc), jnp.float32) for _ in range(depth)]
    scratch += [pltpu.SemaphoreType.DMA for _ in range(depth)]
    if async_scat:
        scratch.append(pltpu.SemaphoreType.DMA)
    if with_deg:
        scratch += [
            pltpu.VMEM_SHARED((_NPAD, 16), jnp.float32),
            pltpu.VMEM((chunk, 16), jnp.float32),
        ]

    def body(*args):
        if with_deg:
            (z_agg, z_deg, ones, table, src, dst, agg_out, deg_out,
             agg_sh, src_v, dst_v, *rest) = args
            bufs, sems = rest[:depth], rest[depth:2 * depth]
            deg_sh, ones_v = rest[2 * depth:]
        else:
            (z_agg, table, src, dst, agg_out,
             agg_sh, src_v, dst_v, *rest) = args
            bufs, sems = rest[:depth], rest[depth:2 * depth]
            sem_s = rest[2 * depth] if async_scat else None
        cid = lax.axis_index("c")
        sid = lax.axis_index("s")
        row0 = sid * _RPT

        pltpu.sync_copy(z_agg, agg_sh.at[pl.ds(row0, _RPT), :])
        if with_deg:
            pltpu.sync_copy(z_deg, deg_sh.at[pl.ds(row0, _RPT), :])
            pltpu.sync_copy(ones, ones_v)
        plsc.subcore_barrier()

        def fire(j, k):
            pltpu.async_copy(table.at[src_v.at[j]], bufs[k], sems[k])

        def drain(k):
            pltpu.make_async_copy(
                table.at[src_v.at[0]], bufs[k], sems[k]).wait()

        def scat(j, k, deg_on):
            pltpu.sync_copy(bufs[k], agg_sh.at[dst_v.at[j]], add=True)
            if with_deg:
                @pl.when(deg_on)
                def _():
                    pltpu.sync_copy(ones_v, deg_sh.at[dst_v.at[j]], add=True)

        for q in range(n_stages):
            pltpu.sync_copy(src.at[cid, sid, q], src_v)
            if core_split:
                pltpu.sync_copy(dst.at[cid, sid, q], dst_v)
            else:
                pltpu.sync_copy(dst.at[sid, q], dst_v)
            deg_on = q // max(1, n_stages // _NC) == cid

            if async_scat:
                def fire_s(j, k):
                    pltpu.async_copy(bufs[k], agg_sh.at[dst_v.at[j]],
                                     sem_s, add=True)

                def drain_s():
                    pltpu.make_async_copy(
                        bufs[0], agg_sh.at[dst_v.at[0]], sem_s).wait()

                fire(0, 0)
                fire(1, 1)
                for i in range(2):
                    fire(i + 2, i + 2)
                    drain(i)
                    fire_s(i, i)

                @pl.loop(0, (nchs - 4) // 4)
                def _(p):
                    for k in range(4):
                        ki = (2 + k) % 4
                        drain_s()
                        fire(4 * p + 4 + k, k)
                        drain(ki)
                        fire_s(4 * p + 2 + k, ki)

                for k in range(2):
                    ki = (2 + k) % 4
                    drain_s()
                    drain(ki)
                    fire_s(nchs - 2 + k, ki)
                drain_s()
                drain_s()
            else:
                for k in range(depth - 1):
                    fire(k, k)

                @pl.loop(0, nchs // depth - 1)
                def _(p):
                    for k in range(depth):
                        j = depth * p + k
                        fire(j + depth - 1, (k + depth - 1) % depth)
                        drain(k)
                        scat(j, k, deg_on)

                fire(nchs - 1, depth - 1)
                for k in range(depth):
                    drain(k)
                    scat(nchs - depth + k, k, deg_on)

        plsc.subcore_barrier()
        pltpu.sync_copy(agg_sh.at[pl.ds(row0, _RPT), :],
                        agg_out.at[cid, pl.ds(row0, _RPT), :])
        if with_deg:
            pltpu.sync_copy(deg_sh.at[pl.ds(row0, _RPT), :],
                            deg_out.at[cid, pl.ds(row0, _RPT), :])

    return pl.kernel(
        body, out_type=out_type, mesh=mesh, scratch_types=scratch,
        name=f"sc_agg_w{wc}",
        compiler_params=pltpu.CompilerParams(use_tc_tiling_on_sc=False))


_sc_agg_l1 = _make_sc_aggregate(128, 4, False, True, _CHUNK, _NCHS, 2)
_sc_agg_l2 = _make_sc_aggregate(32, 4, False, False, _CHUNK, _NCHS, 4,
                                async_scat=True)


def _rowspec(w, bm=_BM):
    return pl.BlockSpec((bm, w), lambda i: (i, 0))


def _pairspec(w, bm=_BM):
    return pl.BlockSpec((_NC, bm, w), lambda i: (0, i, 0))


def _full(shape):
    return pl.BlockSpec(shape, lambda i: (0, 0))


def _tcs_body(x_ref, ws_ref, b_ref, o_ref):
    o_ref[...] = jnp.dot(x_ref[...], ws_ref[...],
                         preferred_element_type=jnp.float32) + b_ref[...]


_tc_self1 = pl.pallas_call(
    _tcs_body,
    grid=(_N // _BM,),
    in_specs=[_rowspec(256), _full((256, 256)), _full((1, 256))],
    out_specs=_rowspec(256),
    out_shape=jax.ShapeDtypeStruct((_N, 256), jnp.float32),
)


def _tca_body(xs_ref, agg_ref, deg_ref, wn0_ref, wn1_ref, ws2_ref, wn2_ref,
              b2_ref, hs_ref, hw_ref):
    deg = deg_ref[0, :, 0:1] + deg_ref[1, :, 0:1]
    r = 1.0 / jnp.maximum(deg, 1.0)
    acc = xs_ref[...]
    acc += jnp.dot(agg_ref[0] * r, wn0_ref[...],
                   preferred_element_type=jnp.float32)
    acc += jnp.dot(agg_ref[1] * r, wn1_ref[...],
                   preferred_element_type=jnp.float32)
    h = jnp.maximum(acc, 0.0)
    hs_ref[...] = jnp.dot(h, ws2_ref[...],
                          preferred_element_type=jnp.float32) + b2_ref[...]
    hw_ref[...] = jnp.dot(h, wn2_ref[...], preferred_element_type=jnp.float32)


_tc_mid = pl.pallas_call(
    _tca_body,
    grid=(_N // _BM,),
    in_specs=[_rowspec(256), _pairspec(128), _pairspec(16),
              _full((128, 256)), _full((128, 256)),
              _full((256, 64)), _full((256, 64)), _full((1, 64))],
    out_specs=[_rowspec(64), _rowspec(64)],
    out_shape=[jax.ShapeDtypeStruct((_N, 64), jnp.float32),
               jax.ShapeDtypeStruct((_N, 64), jnp.float32)],
)


def _tcf_body(hs_ref, agg_ref, deg_ref, o_ref):
    deg = deg_ref[0, :, 0:1] + deg_ref[1, :, 0:1]
    r = 1.0 / jnp.maximum(deg, 1.0)
    o_ref[...] = hs_ref[...] + jnp.concatenate(
        [agg_ref[0] * r, agg_ref[1] * r], axis=1)


_tc_final = pl.pallas_call(
    _tcf_body,
    grid=(_N // _BMF,),
    in_specs=[_rowspec(64, _BMF), _pairspec(32, _BMF), _pairspec(16, _BMF)],
    out_specs=_rowspec(64, _BMF),
    out_shape=jax.ShapeDtypeStruct((_N, 64), jnp.float32),
)


def kernel(x, edge_index, W_self1, W_neigh1, b1, W_self2, W_neigh2, b2):
    src = edge_index[0].astype(jnp.int32)
    dst = edge_index[1].astype(jnp.int32)
    pad = _EPAD - _E
    srcp = jnp.concatenate([src, jnp.zeros((pad,), jnp.int32)])
    pad_dst = _N + jax.lax.rem(jnp.arange(pad, dtype=jnp.int32),
                               jnp.int32(_NPAD - _N))
    dstp = jnp.concatenate([dst, pad_dst])
    base = srcp * 2
    src3 = jnp.stack([base, base + 1]).reshape(_NC, _NS, 4, _NCHS, _CHUNK)
    dst3 = dstp.reshape(_NS, 4, _NCHS, _CHUNK)

    z1 = jnp.zeros((_RPT, 128), jnp.float32)
    zd = jnp.zeros((_RPT, 16), jnp.float32)
    ones = jnp.ones((_CHUNK, 16), jnp.float32)
    table1 = x.reshape(2 * _N, 128)
    agg1, deg = _sc_agg_l1(z1, zd, ones, table1, src3, dst3)

    xs = _tc_self1(x, W_self1, b1.reshape(1, 256))
    hs, hw = _tc_mid(xs, agg1, deg, W_neigh1[:128], W_neigh1[128:],
                     W_self2, W_neigh2, b2.reshape(1, 64))

    z2 = jnp.zeros((_RPT, 32), jnp.float32)
    table2 = hw.reshape(2 * _N, 32)
    agg2 = _sc_agg_l2(z2, table2, src3, dst3)

    return _tc_final(hs, agg2, deg)
